# baseline (speedup 1.0000x reference)
import functools
import jax
import jax.numpy as jnp
from jax import lax
from jax.experimental import pallas as pl
from jax.experimental.pallas import tpu as pltpu, tpu_sc as plsc

_N = 1000
_B = 16384
_CHUNK = 64


@functools.cache
def _make_sc_kernel():
    info = plsc.get_sparse_core_info()
    NC, NS, L = info.num_cores, info.num_subcores, info.num_lanes
    NW = NC * NS
    b_per_w = _B // NW
    n_chunks = b_per_w // _CHUNK
    mesh = plsc.VectorSubcoreMesh(core_axis_name="c", subcore_axis_name="s")

    @functools.partial(
        pl.kernel, mesh=mesh,
        compiler_params=pltpu.CompilerParams(
            use_tc_tiling_on_sc=False, needs_layout_passes=False),
        out_type=jax.ShapeDtypeStruct((_B, _N), jnp.float32),
        scratch_types=[
            pltpu.VMEM((b_per_w,), jnp.int32),
            pltpu.VMEM((_CHUNK, _N), jnp.float32),
        ],
    )
    def k(idx_hbm, out_hbm, idx_v, buf):
        wid = lax.axis_index("s") * NC + lax.axis_index("c")
        base = wid * b_per_w
        pltpu.sync_copy(idx_hbm.at[pl.ds(base, b_per_w)], idx_v)
        zeros = jnp.zeros((L,), jnp.float32)
        ones = jnp.ones((L,), jnp.float32)
        lane = lax.iota(jnp.int32, L)

        def zrow(r, carry):
            for j in range(_N // L):
                buf[r, pl.ds(j * L, L)] = zeros
            buf[r, pl.ds(_N - L, L)] = zeros
            return carry

        lax.fori_loop(0, _CHUNK, zrow, 0)

        for c in range(n_chunks):
            for g in range(_CHUNK // L):
                rows = g * L + lane
                cols = idx_v[pl.ds(c * _CHUNK + g * L, L)]
                plsc.store_scatter(buf, [rows, cols], ones)
            pltpu.sync_copy(buf, out_hbm.at[pl.ds(base + c * _CHUNK, _CHUNK)])
            if c != n_chunks - 1:
                for g in range(_CHUNK // L):
                    rows = g * L + lane
                    cols = idx_v[pl.ds(c * _CHUNK + g * L, L)]
                    plsc.store_scatter(buf, [rows, cols], zeros)

    return k


def kernel(inputs, indices):
    del inputs
    return _make_sc_kernel()(indices)


# speedup vs baseline: 1.6031x; 1.6031x over previous
import functools
import jax
import jax.numpy as jnp
from jax import lax
from jax.experimental import pallas as pl
from jax.experimental.pallas import tpu as pltpu, tpu_sc as plsc

_N = 1000
_B = 16384
_CHUNK = 64


@functools.cache
def _make_sc_kernel():
    info = plsc.get_sparse_core_info()
    NC, NS, L = info.num_cores, info.num_subcores, info.num_lanes
    NW = NC * NS
    b_per_w = _B // NW
    n_chunks = b_per_w // _CHUNK
    mesh = plsc.VectorSubcoreMesh(core_axis_name="c", subcore_axis_name="s")

    @functools.partial(
        pl.kernel, mesh=mesh,
        compiler_params=pltpu.CompilerParams(
            use_tc_tiling_on_sc=True, needs_layout_passes=False),
        out_type=jax.ShapeDtypeStruct((_B, _N), jnp.float32),
        scratch_types=[
            pltpu.VMEM((b_per_w,), jnp.int32),
            pltpu.VMEM((_CHUNK, _N), jnp.float32),
        ],
    )
    def k(idx_hbm, out_hbm, idx_v, buf):
        wid = lax.axis_index("s") * NC + lax.axis_index("c")
        base = wid * b_per_w
        pltpu.sync_copy(idx_hbm.at[pl.ds(base, b_per_w)], idx_v)
        zeros = jnp.zeros((L,), jnp.float32)
        ones = jnp.ones((L,), jnp.float32)
        lane = lax.iota(jnp.int32, L)

        def zrow(r, carry):
            for j in range(_N // L):
                buf[r, pl.ds(j * L, L)] = zeros
            buf[r, pl.ds(_N - L, L)] = zeros
            return carry

        lax.fori_loop(0, _CHUNK, zrow, 0)

        for c in range(n_chunks):
            for g in range(_CHUNK // L):
                rows = g * L + lane
                cols = idx_v[pl.ds(c * _CHUNK + g * L, L)]
                plsc.store_scatter(buf, [rows, cols], ones)
            pltpu.sync_copy(buf, out_hbm.at[pl.ds(base + c * _CHUNK, _CHUNK)])
            if c != n_chunks - 1:
                for g in range(_CHUNK // L):
                    rows = g * L + lane
                    cols = idx_v[pl.ds(c * _CHUNK + g * L, L)]
                    plsc.store_scatter(buf, [rows, cols], zeros)

    return k


def kernel(inputs, indices):
    del inputs
    return _make_sc_kernel()(indices)


# speedup vs baseline: 3.6067x; 2.2499x over previous
import functools
import jax
import jax.numpy as jnp
from jax import lax
from jax.experimental import pallas as pl
from jax.experimental.pallas import tpu as pltpu, tpu_sc as plsc

_N = 1000
_B = 16384
_CHUNK_COLS = 128


@functools.cache
def _make_sc_kernel():
    info = plsc.get_sparse_core_info()
    NC, NS, L = info.num_cores, info.num_subcores, info.num_lanes
    NW = NC * NS
    cols_per_w = _B // NW
    n_chunks = cols_per_w // _CHUNK_COLS
    mesh = plsc.VectorSubcoreMesh(core_axis_name="c", subcore_axis_name="s")

    @functools.partial(
        pl.kernel, mesh=mesh,
        compiler_params=pltpu.CompilerParams(
            use_tc_tiling_on_sc=True, needs_layout_passes=False),
        out_type=jax.ShapeDtypeStruct((_N, _B), jnp.float32),
        scratch_types=[
            pltpu.VMEM((cols_per_w,), jnp.int32),
            pltpu.VMEM((_N, _CHUNK_COLS), jnp.float32),
        ],
    )
    def k(idx_hbm, out_hbm, idx_v, buf):
        wid = lax.axis_index("s") * NC + lax.axis_index("c")
        base = wid * cols_per_w
        pltpu.sync_copy(idx_hbm.at[pl.ds(base, cols_per_w)], idx_v)
        zeros = jnp.zeros((L,), jnp.float32)
        ones = jnp.ones((L,), jnp.float32)
        lane = lax.iota(jnp.int32, L)

        def zrow(r, carry):
            for j in range(_CHUNK_COLS // L):
                buf[r, pl.ds(j * L, L)] = zeros
            return carry

        lax.fori_loop(0, _N, zrow, 0)

        for c in range(n_chunks):
            for g in range(_CHUNK_COLS // L):
                rows = idx_v[pl.ds(c * _CHUNK_COLS + g * L, L)]
                cols = g * L + lane
                plsc.store_scatter(buf, [rows, cols], ones)
            pltpu.sync_copy(
                buf, out_hbm.at[:, pl.ds(base + c * _CHUNK_COLS, _CHUNK_COLS)])
            if c != n_chunks - 1:
                for g in range(_CHUNK_COLS // L):
                    rows = idx_v[pl.ds(c * _CHUNK_COLS + g * L, L)]
                    cols = g * L + lane
                    plsc.store_scatter(buf, [rows, cols], zeros)

    return k


def kernel(inputs, indices):
    del inputs
    return _make_sc_kernel()(indices).T


# speedup vs baseline: 3.6781x; 1.0198x over previous
import functools
import jax
import jax.numpy as jnp
from jax import lax
from jax.experimental import pallas as pl
from jax.experimental.pallas import tpu as pltpu, tpu_sc as plsc

_N = 1000
_B = 16384
_CC = 128
_RT = 504
_RB = _N - _RT


@functools.cache
def _make_sc_kernel():
    info = plsc.get_sparse_core_info()
    NC, NS, L = info.num_cores, info.num_subcores, info.num_lanes
    NW = NC * NS
    cols_per_w = _B // NW
    n_chunks = cols_per_w // _CC
    mesh = plsc.VectorSubcoreMesh(core_axis_name="c", subcore_axis_name="s")

    @functools.partial(
        pl.kernel, mesh=mesh,
        compiler_params=pltpu.CompilerParams(
            use_tc_tiling_on_sc=True, needs_layout_passes=False),
        out_type=jax.ShapeDtypeStruct((_N, _B), jnp.float32),
        scratch_types=[
            pltpu.VMEM((cols_per_w,), jnp.int32),
            pltpu.VMEM((_RT, _CC), jnp.float32),
            pltpu.VMEM((_RB, _CC), jnp.float32),
            pltpu.SemaphoreType.DMA,
            pltpu.SemaphoreType.DMA,
        ],
    )
    def k(idx_hbm, out_hbm, idx_v, buf_t, buf_b, sem_t, sem_b):
        wid = lax.axis_index("s") * NC + lax.axis_index("c")
        base = wid * cols_per_w
        pltpu.sync_copy(idx_hbm.at[pl.ds(base, cols_per_w)], idx_v)
        zeros = jnp.zeros((L,), jnp.float32)
        ones = jnp.ones((L,), jnp.float32)
        lane = lax.iota(jnp.int32, L)

        def zfill(buf, nrows):
            def zrow(r, carry):
                for j in range(_CC // L):
                    buf[r, pl.ds(j * L, L)] = zeros
                return carry
            lax.fori_loop(0, nrows, zrow, 0)

        def scatter_top(c, val):
            for g in range(_CC // L):
                rows = idx_v[pl.ds(c * _CC + g * L, L)]
                plsc.store_scatter(
                    buf_t, [jnp.minimum(rows, _RT - 1), g * L + lane], val,
                    mask=rows < _RT)

        def scatter_bot(c, val):
            for g in range(_CC // L):
                rows = idx_v[pl.ds(c * _CC + g * L, L)]
                plsc.store_scatter(
                    buf_b, [jnp.maximum(rows - _RT, 0), g * L + lane], val,
                    mask=rows >= _RT)

        def dma_t(c):
            return pltpu.async_copy(
                buf_t, out_hbm.at[pl.ds(0, _RT),
                                  pl.ds(base + c * _CC, _CC)], sem_t)

        def dma_b(c):
            return pltpu.async_copy(
                buf_b, out_hbm.at[pl.ds(_RT, _RB),
                                  pl.ds(base + c * _CC, _CC)], sem_b)

        zfill(buf_t, _RT)
        scatter_top(0, ones)
        ht = dma_t(0)
        zfill(buf_b, _RB)
        scatter_bot(0, ones)
        hb = dma_b(0)
        for c in range(1, n_chunks):
            ht.wait()
            scatter_top(c - 1, zeros)
            scatter_top(c, ones)
            ht = dma_t(c)
            hb.wait()
            scatter_bot(c - 1, zeros)
            scatter_bot(c, ones)
            hb = dma_b(c)
        ht.wait()
        hb.wait()

    return k


def kernel(inputs, indices):
    del inputs
    return _make_sc_kernel()(indices).T
